# baseline (speedup 1.0000x reference)
import functools

import jax
import jax.numpy as jnp
from jax import lax
from jax.experimental import pallas as pl
from jax.experimental.pallas import tpu as pltpu
from jax.experimental.pallas import tpu_sc as plsc

EMBED = 128
NC = 2
NS = 16
NW = NC * NS
CHUNK = 128


NBUF = 5
LOOKAHEAD = 4


@functools.partial(jax.jit, static_argnames=("n_chunk",))
def _gather(idx, table, n_chunk):
    mesh = plsc.VectorSubcoreMesh(core_axis_name="c", subcore_axis_name="s")

    @functools.partial(
        pl.kernel,
        out_type=jax.ShapeDtypeStruct((NW * n_chunk * CHUNK, EMBED), jnp.float32),
        mesh=mesh,
        scratch_types=[
            pltpu.VMEM((n_chunk, CHUNK), jnp.int32),
            [pltpu.VMEM((CHUNK, EMBED), jnp.float32) for _ in range(NBUF)],
            [pltpu.SemaphoreType.DMA for _ in range(NBUF)],
            [pltpu.SemaphoreType.DMA for _ in range(NBUF)],
        ],
    )
    def body(idx_hbm, table_hbm, out_hbm, idx_v, rows, gsem, osem):
        wid = lax.axis_index("s") * NC + lax.axis_index("c")
        base = wid * (n_chunk * CHUNK)
        pltpu.sync_copy(idx_hbm.at[wid], idx_v)

        def fire_gather(j, b):
            pltpu.async_copy(table_hbm.at[idx_v.at[j]], rows[b], gsem[b])

        def wait_gather(b):
            pltpu.make_async_copy(table_hbm.at[idx_v.at[0]], rows[b], gsem[b]).wait()

        def fire_store(j, b):
            pltpu.async_copy(rows[b], out_hbm.at[pl.ds(base + j * CHUNK, CHUNK)], osem[b])

        def wait_store(b):
            pltpu.make_async_copy(rows[b], out_hbm.at[pl.ds(base, CHUNK)], osem[b]).wait()

        for b in range(LOOKAHEAD):
            fire_gather(b, b)

        @pl.loop(0, n_chunk, step=NBUF)
        def _(j0):
            for b in range(NBUF):
                j = j0 + b
                wait_gather(b)
                fire_store(j, b)
                b2 = (b + LOOKAHEAD) % NBUF

                @pl.when((j >= NBUF - LOOKAHEAD) & (j + LOOKAHEAD < n_chunk))
                def _():
                    wait_store(b2)

                @pl.when(j + LOOKAHEAD < n_chunk)
                def _():
                    fire_gather(j + LOOKAHEAD, b2)

        for b in range(NBUF):
            wait_store(b)

    return body(idx, table)


def kernel(sequence, table):
    b, l = sequence.shape
    total = b * l
    n_chunk = total // (NW * CHUNK)
    idx = sequence.reshape(NW, n_chunk, CHUNK).astype(jnp.int32)
    out = _gather(idx, table, n_chunk)
    return out.reshape(b, l, EMBED)


# speedup vs baseline: 1.0067x; 1.0067x over previous
import functools

import jax
import jax.numpy as jnp
from jax import lax
from jax.experimental import pallas as pl
from jax.experimental.pallas import tpu as pltpu
from jax.experimental.pallas import tpu_sc as plsc

EMBED = 128
NC = 2
NS = 16
NW = NC * NS
CHUNK = 64


NBUF = 10
LOOKAHEAD = 6


@functools.partial(jax.jit, static_argnames=("n_chunk",))
def _gather(idx, table, n_chunk):
    mesh = plsc.VectorSubcoreMesh(core_axis_name="c", subcore_axis_name="s")

    @functools.partial(
        pl.kernel,
        out_type=jax.ShapeDtypeStruct((NW * n_chunk * CHUNK, EMBED), jnp.float32),
        mesh=mesh,
        scratch_types=[
            pltpu.VMEM((n_chunk, CHUNK), jnp.int32),
            [pltpu.VMEM((CHUNK, EMBED), jnp.float32) for _ in range(NBUF)],
            [pltpu.SemaphoreType.DMA for _ in range(NBUF)],
            [pltpu.SemaphoreType.DMA for _ in range(NBUF)],
        ],
    )
    def body(idx_hbm, table_hbm, out_hbm, idx_v, rows, gsem, osem):
        wid = lax.axis_index("s") * NC + lax.axis_index("c")
        base = wid * (n_chunk * CHUNK)
        pltpu.sync_copy(idx_hbm.at[wid], idx_v)

        def fire_gather(j, b):
            pltpu.async_copy(table_hbm.at[idx_v.at[j]], rows[b], gsem[b])

        def wait_gather(b):
            pltpu.make_async_copy(table_hbm.at[idx_v.at[0]], rows[b], gsem[b]).wait()

        def fire_store(j, b):
            pltpu.async_copy(rows[b], out_hbm.at[pl.ds(base + j * CHUNK, CHUNK)], osem[b])

        def wait_store(b):
            pltpu.make_async_copy(rows[b], out_hbm.at[pl.ds(base, CHUNK)], osem[b]).wait()

        for b in range(LOOKAHEAD):
            fire_gather(b, b)

        @pl.loop(0, n_chunk, step=NBUF)
        def _(j0):
            for b in range(NBUF):
                j = j0 + b
                wait_gather(b)
                fire_store(j, b)
                b2 = (b + LOOKAHEAD) % NBUF

                @pl.when((j >= NBUF - LOOKAHEAD) & (j + LOOKAHEAD < n_chunk))
                def _():
                    wait_store(b2)

                @pl.when(j + LOOKAHEAD < n_chunk)
                def _():
                    fire_gather(j + LOOKAHEAD, b2)

        for b in range(NBUF):
            wait_store(b)

    return body(idx, table)


def kernel(sequence, table):
    b, l = sequence.shape
    total = b * l
    n_chunk = total // (NW * CHUNK)
    idx = sequence.reshape(NW, n_chunk, CHUNK).astype(jnp.int32)
    out = _gather(idx, table, n_chunk)
    return out.reshape(b, l, EMBED)


# speedup vs baseline: 1.0102x; 1.0035x over previous
import functools

import jax
import jax.numpy as jnp
from jax import lax
from jax.experimental import pallas as pl
from jax.experimental.pallas import tpu as pltpu
from jax.experimental.pallas import tpu_sc as plsc

EMBED = 128
NC = 2
NS = 16
NW = NC * NS
CHUNK = 64


NBUF = 10
LOOKAHEAD = 8


@functools.partial(jax.jit, static_argnames=("n_chunk",))
def _gather(idx, table, n_chunk):
    mesh = plsc.VectorSubcoreMesh(core_axis_name="c", subcore_axis_name="s")

    @functools.partial(
        pl.kernel,
        out_type=jax.ShapeDtypeStruct((NW * n_chunk * CHUNK, EMBED), jnp.float32),
        mesh=mesh,
        scratch_types=[
            pltpu.VMEM((n_chunk, CHUNK), jnp.int32),
            [pltpu.VMEM((CHUNK, EMBED), jnp.float32) for _ in range(NBUF)],
            [pltpu.SemaphoreType.DMA for _ in range(NBUF)],
            [pltpu.SemaphoreType.DMA for _ in range(NBUF)],
        ],
    )
    def body(idx_hbm, table_hbm, out_hbm, idx_v, rows, gsem, osem):
        wid = lax.axis_index("s") * NC + lax.axis_index("c")
        base = wid * (n_chunk * CHUNK)
        pltpu.sync_copy(idx_hbm.at[wid], idx_v)

        def fire_gather(j, b):
            pltpu.async_copy(table_hbm.at[idx_v.at[j]], rows[b], gsem[b])

        def wait_gather(b):
            pltpu.make_async_copy(table_hbm.at[idx_v.at[0]], rows[b], gsem[b]).wait()

        def fire_store(j, b):
            pltpu.async_copy(rows[b], out_hbm.at[pl.ds(base + j * CHUNK, CHUNK)], osem[b])

        def wait_store(b):
            pltpu.make_async_copy(rows[b], out_hbm.at[pl.ds(base, CHUNK)], osem[b]).wait()

        for b in range(LOOKAHEAD):
            fire_gather(b, b)

        @pl.loop(0, n_chunk, step=NBUF)
        def _(j0):
            for b in range(NBUF):
                j = j0 + b
                wait_gather(b)
                fire_store(j, b)
                b2 = (b + LOOKAHEAD) % NBUF

                @pl.when((j >= NBUF - LOOKAHEAD) & (j + LOOKAHEAD < n_chunk))
                def _():
                    wait_store(b2)

                @pl.when(j + LOOKAHEAD < n_chunk)
                def _():
                    fire_gather(j + LOOKAHEAD, b2)

        for b in range(NBUF):
            wait_store(b)

    return body(idx, table)


def kernel(sequence, table):
    b, l = sequence.shape
    total = b * l
    n_chunk = total // (NW * CHUNK)
    idx = sequence.reshape(NW, n_chunk, CHUNK).astype(jnp.int32)
    out = _gather(idx, table, n_chunk)
    return out.reshape(b, l, EMBED)
